# baseline (speedup 1.0000x reference)
import functools

import jax
import jax.numpy as jnp
from jax.experimental import pallas as pl
from jax.experimental.pallas import tpu as pltpu
from jax.experimental.pallas import tpu_sc as plsc

D_MODEL = 1024
N_HEADS = 16
D_FF = 4096
N_EXPERTS = 8
S = 2048
N_ASSIGN = 2 * S
ROW_T = 256
MAX_TILES = 24
MAX_ROWS = MAX_TILES * ROW_T
Q_TILE = 512
S_TILE = 256


def _attn_body(xq_ref, xf_ref, w_ref, b_ref, o_ref, kv_ref):
    qt = pl.program_id(1)
    w = w_ref[0]
    b = b_ref[0, 0]

    @pl.when(qt == 0)
    def _():
        xf = xf_ref[...]
        kv_ref[:, :128] = (jnp.dot(xf, w[:, 128:256],
                                   preferred_element_type=jnp.float32)
                           + b[128:256]).astype(jnp.bfloat16)
        kv_ref[:, 128:] = (jnp.dot(xf, w[:, 256:384],
                                   preferred_element_type=jnp.float32)
                           + b[256:384]).astype(jnp.bfloat16)

    xq = xq_ref[...]
    q2 = (jnp.dot(xq, w[:, :128], preferred_element_type=jnp.float32)
          + b[:128]).astype(jnp.bfloat16)
    k2 = kv_ref[:, :128]
    v2 = kv_ref[:, 128:]
    outs = []
    for h in range(2):
        q = q2[:, 64 * h:64 * (h + 1)]
        k = k2[:, 64 * h:64 * (h + 1)]
        v = v2[:, 64 * h:64 * (h + 1)]
        s = jax.lax.dot_general(q, k, (((1,), (1,)), ((), ())),
                                preferred_element_type=jnp.float32) * 0.125
        m = jnp.max(s, axis=1, keepdims=True)
        e = jnp.exp(s - m)
        aw = (e / jnp.sum(e, axis=1, keepdims=True)).astype(jnp.bfloat16)
        outs.append(jnp.dot(aw, v, preferred_element_type=jnp.float32))
    o_ref[...] = jnp.concatenate(outs, axis=1)


def _attention(x2d, Wqkv, bqkv):
    return pl.pallas_call(
        _attn_body,
        grid=(N_HEADS // 2, S // Q_TILE),
        in_specs=[
            pl.BlockSpec((Q_TILE, D_MODEL), lambda p, q: (q, 0)),
            pl.BlockSpec((S, D_MODEL), lambda p, q: (0, 0)),
            pl.BlockSpec((1, D_MODEL, 384), lambda p, q: (p, 0, 0)),
            pl.BlockSpec((1, 1, 384), lambda p, q: (p, 0, 0)),
        ],
        out_specs=pl.BlockSpec((Q_TILE, 128), lambda p, q: (q, p)),
        out_shape=jax.ShapeDtypeStruct((S, D_MODEL), jnp.float32),
        scratch_shapes=[pltpu.VMEM((S, 256), jnp.bfloat16)],
    )(x2d, x2d, Wqkv, bqkv)


def _ln(v, g, b):
    mu = jnp.mean(v, axis=-1, keepdims=True)
    var = jnp.mean((v - mu) ** 2, axis=-1, keepdims=True)
    return (v - mu) * jax.lax.rsqrt(var + 1e-5) * g + b


def _postattn_body(a_ref, x_ref, wo_ref, bo_ref, g_ref, b_ref, wr_ref,
                   h_ref, hb_ref, lg_ref):
    a = a_ref[...].astype(jnp.bfloat16)
    proj = jnp.dot(a, wo_ref[...], preferred_element_type=jnp.float32)
    h = _ln(x_ref[...] + proj + bo_ref[0], g_ref[0], b_ref[0])
    h_ref[...] = h
    hb = h.astype(jnp.bfloat16)
    hb_ref[...] = hb
    lg_ref[...] = jnp.dot(hb, wr_ref[...], preferred_element_type=jnp.float32)


def _postattn(attn, x2d, Wo, bo_r, ln_g, ln_b, Wr_pad):
    return pl.pallas_call(
        _postattn_body,
        grid=(S // S_TILE,),
        in_specs=[
            pl.BlockSpec((S_TILE, D_MODEL), lambda t: (t, 0)),
            pl.BlockSpec((S_TILE, D_MODEL), lambda t: (t, 0)),
            pl.BlockSpec((D_MODEL, D_MODEL), lambda t: (0, 0)),
            pl.BlockSpec((8, D_MODEL), lambda t: (0, 0)),
            pl.BlockSpec((8, D_MODEL), lambda t: (0, 0)),
            pl.BlockSpec((8, D_MODEL), lambda t: (0, 0)),
            pl.BlockSpec((D_MODEL, 128), lambda t: (0, 0)),
        ],
        out_specs=[
            pl.BlockSpec((S_TILE, D_MODEL), lambda t: (t, 0)),
            pl.BlockSpec((S_TILE, D_MODEL), lambda t: (t, 0)),
            pl.BlockSpec((S_TILE, 128), lambda t: (t, 0)),
        ],
        out_shape=[
            jax.ShapeDtypeStruct((S, D_MODEL), jnp.float32),
            jax.ShapeDtypeStruct((S, D_MODEL), jnp.bfloat16),
            jax.ShapeDtypeStruct((S, 128), jnp.float32),
        ],
    )(attn, x2d, Wo, bo_r, ln_g, ln_b, Wr_pad)


def _top2(l8):
    lane = jax.lax.broadcasted_iota(jnp.int32, l8.shape, 1)
    m = jnp.max(l8, axis=1, keepdims=True)
    p = jnp.exp(l8 - m)
    m1 = jnp.max(p, axis=1, keepdims=True)
    a0 = jnp.min(jnp.where(p == m1, lane, N_EXPERTS), axis=1, keepdims=True)
    p2 = jnp.where(lane == a0, -1.0, p)
    m2 = jnp.max(p2, axis=1, keepdims=True)
    a1 = jnp.min(jnp.where(p2 == m2, lane, N_EXPERTS), axis=1, keepdims=True)
    tot = m1 + m2
    return m1 / tot, m2 / tot, a0, a1


_NEG = -1e30


def _route_meta_body(lg_ref, gates_ref, pos_ref, te_ref, e_scr, rank_scr):
    lane = jax.lax.broadcasted_iota(jnp.int32, (S, 128), 1)
    l8 = jnp.where(lane < N_EXPERTS, lg_ref[...], _NEG)
    g0, g1, a0, a1 = _top2(l8)
    gates_ref[...] = jnp.where(lane == 0, g0, 0.0) + jnp.where(lane == 1, g1, 0.0)

    e_col = jnp.concatenate([a0, a1], axis=0)
    lane4 = jax.lax.broadcasted_iota(jnp.int32, (N_ASSIGN, 128), 1)
    E = (lane4 == e_col).astype(jnp.float32)
    e_scr[...] = E

    r = jax.lax.broadcasted_iota(jnp.int32, (128, 128), 0)
    c = jax.lax.broadcasted_iota(jnp.int32, (128, 128), 1)
    T128 = (r >= c).astype(jnp.float32)

    def chunk1(i, _):
        Ec = e_scr[pl.ds(i * 128, 128), :]
        C = jnp.dot(T128, Ec, preferred_element_type=jnp.float32)
        rank_scr[pl.ds(i * 128, 128), :] = C - Ec
        return 0

    jax.lax.fori_loop(0, N_ASSIGN // 128, chunk1, 0)

    nch = N_ASSIGN // 128
    Mrow = jax.lax.broadcasted_iota(jnp.int32, (nch, N_ASSIGN), 0)
    Mcol = jax.lax.broadcasted_iota(jnp.int32, (nch, N_ASSIGN), 1) // 128
    M = (Mrow == Mcol).astype(jnp.float32)
    Sm = jnp.dot(M, E, preferred_element_type=jnp.float32)

    r32 = jax.lax.broadcasted_iota(jnp.int32, (nch, nch), 0)
    c32 = jax.lax.broadcasted_iota(jnp.int32, (nch, nch), 1)
    T32s = (r32 > c32).astype(jnp.float32)
    O = jnp.dot(T32s, Sm, preferred_element_type=jnp.float32)
    MTrow = jax.lax.broadcasted_iota(jnp.int32, (N_ASSIGN, nch), 0) // 128
    MTcol = jax.lax.broadcasted_iota(jnp.int32, (N_ASSIGN, nch), 1)
    MT = (MTrow == MTcol).astype(jnp.float32)
    Oh = jnp.floor(O * (1.0 / 256.0))
    Ol = O - 256.0 * Oh
    OB = (256.0 * jnp.dot(MT, Oh, preferred_element_type=jnp.float32)
          + jnp.dot(MT, Ol, preferred_element_type=jnp.float32))

    tot = jnp.sum(Sm, axis=0, keepdims=True)
    pc = jnp.ceil(tot / ROW_T) * ROW_T
    cu = (jax.lax.broadcasted_iota(jnp.int32, (128, 128), 0) <
          jax.lax.broadcasted_iota(jnp.int32, (128, 128), 1)).astype(jnp.float32)
    pc8 = jnp.broadcast_to(pc * (1.0 / ROW_T), (8, 128))
    off = ROW_T * jnp.dot(pc8, cu, preferred_element_type=jnp.float32)[0:1, :]

    posm = rank_scr[...] + OB + off
    pos_col = jnp.sum(posm * E, axis=1, keepdims=True)
    pos2 = 2.0 * jnp.broadcast_to(pos_col, (N_ASSIGN, 128))
    pos_ref[...] = (pos2 + (lane4 == 1).astype(jnp.float32)).astype(jnp.int32)

    end = off + pc
    t_col = (jax.lax.broadcasted_iota(jnp.int32, (32, 128), 0) *
             ROW_T).astype(jnp.float32)
    lane32 = jax.lax.broadcasted_iota(jnp.int32, (32, 128), 1)
    fin = jnp.where((t_col >= end) & (lane32 < N_EXPERTS), 1.0, 0.0)
    te = jnp.minimum(jnp.sum(fin, axis=1, keepdims=True),
                     float(N_EXPERTS - 1))
    te_ref[...] = jnp.broadcast_to(te, (32, 128)).astype(jnp.int32)


def _route_meta(logits):
    return pl.pallas_call(
        _route_meta_body,
        grid=(1,),
        in_specs=[pl.BlockSpec((S, 128), lambda i: (0, 0))],
        out_specs=[
            pl.BlockSpec((S, 128), lambda i: (0, 0)),
            pl.BlockSpec((N_ASSIGN, 128), lambda i: (0, 0)),
            pl.BlockSpec((32, 128), lambda i: (0, 0)),
        ],
        out_shape=[
            jax.ShapeDtypeStruct((S, 128), jnp.float32),
            jax.ShapeDtypeStruct((N_ASSIGN, 128), jnp.int32),
            jax.ShapeDtypeStruct((32, 128), jnp.int32),
        ],
        scratch_shapes=[
            pltpu.VMEM((N_ASSIGN, 128), jnp.float32),
            pltpu.VMEM((N_ASSIGN, 128), jnp.float32),
        ],
    )(logits)


_SC_WIN = 128
_SC_D = 256


def _to_words(a, n):
    return jax.lax.bitcast_convert_type(
        a.reshape(n, _SC_D, 2), jnp.int32)


def _from_words(w, n_rows):
    return jax.lax.bitcast_convert_type(
        w, jnp.bfloat16).reshape(n_rows, D_MODEL)


def _sc_scatter_rows(src, idx, n_out):
    mesh = plsc.VectorSubcoreMesh(core_axis_name="core",
                                  subcore_axis_name="subcore")
    n, d = src.shape

    @functools.partial(pl.kernel,
                       out_type=jax.ShapeDtypeStruct((n_out, d), src.dtype),
                       mesh=mesh, scratch_types=[])
    def kern(x_hbm, i_hbm, o_hbm):
        def body(x_vmem, i_vmem):
            pltpu.sync_copy(x_vmem, o_hbm.at[i_vmem.at[0]])

        pltpu.emit_pipeline(
            body,
            grid=(n // _SC_WIN,),
            in_specs=[
                pl.BlockSpec((_SC_WIN, d), lambda i: (i, 0)),
                pl.BlockSpec((1, _SC_WIN), lambda i: (0, i)),
            ],
            out_specs=[],
            core_axis_name=("core", "subcore"),
            dimension_semantics=(pltpu.PARALLEL,),
        )(x_hbm, i_hbm)

    return kern(src, idx)


def _sc_gather_rows(data, idx):
    mesh = plsc.VectorSubcoreMesh(core_axis_name="core",
                                  subcore_axis_name="subcore")
    d = data.shape[1]
    n = idx.shape[1]

    @functools.partial(pl.kernel,
                       out_type=jax.ShapeDtypeStruct((n, d), data.dtype),
                       mesh=mesh, scratch_types=[])
    def kern(x_hbm, i_hbm, o_hbm):
        def body(i_vmem, o_vmem):
            pltpu.sync_copy(x_hbm.at[i_vmem.at[0]], o_vmem)

        pltpu.emit_pipeline(
            body,
            grid=(n // _SC_WIN,),
            in_specs=[pl.BlockSpec((1, _SC_WIN), lambda i: (0, i))],
            out_specs=[pl.BlockSpec((_SC_WIN, d), lambda i: (i, 0))],
            core_axis_name=("core", "subcore"),
            dimension_semantics=(pltpu.PARALLEL,),
        )(i_hbm, o_hbm)

    return kern(data, idx)


def _ffn_body(te_ref, hg_ref, w1_ref, b1_ref, w2_ref, b2_ref, og_ref):
    hg = hg_ref[...]
    h1 = jnp.dot(hg, w1_ref[0], preferred_element_type=jnp.float32)
    h1 = jnp.maximum(h1 + b1_ref[0, 0], 0.0).astype(jnp.bfloat16)
    og = jnp.dot(h1, w2_ref[0], preferred_element_type=jnp.float32)
    og_ref[...] = (og + b2_ref[0, 0]).astype(jnp.bfloat16)


def _moe_ffn(te, hg, W1b, b1r, W2b, b2r):
    grid_spec = pltpu.PrefetchScalarGridSpec(
        num_scalar_prefetch=1,
        grid=(MAX_TILES,),
        in_specs=[
            pl.BlockSpec((ROW_T, D_MODEL), lambda t, te: (t, 0)),
            pl.BlockSpec((1, D_MODEL, D_FF), lambda t, te: (te[t], 0, 0)),
            pl.BlockSpec((1, 1, D_FF), lambda t, te: (te[t], 0, 0)),
            pl.BlockSpec((1, D_FF, D_MODEL), lambda t, te: (te[t], 0, 0)),
            pl.BlockSpec((1, 1, D_MODEL), lambda t, te: (te[t], 0, 0)),
        ],
        out_specs=pl.BlockSpec((ROW_T, D_MODEL), lambda t, te: (t, 0)),
    )
    return pl.pallas_call(
        _ffn_body,
        grid_spec=grid_spec,
        out_shape=jax.ShapeDtypeStruct((MAX_ROWS, D_MODEL), jnp.bfloat16),
    )(te, hg, W1b, b1r, W2b, b2r)


def _final_body(h_ref, c0_ref, c1_ref, gt_ref, g_ref, b_ref, wr_ref, br_ref,
                x2_ref, aux_ref, accf_ref, accp_ref):
    t = pl.program_id(0)

    @pl.when(t == 0)
    def _():
        accf_ref[...] = jnp.zeros_like(accf_ref)
        accp_ref[...] = jnp.zeros_like(accp_ref)

    g0 = gt_ref[:, 0:1].astype(jnp.bfloat16).astype(jnp.float32)
    g1 = gt_ref[:, 1:2].astype(jnp.bfloat16).astype(jnp.float32)
    moe = g0 * c0_ref[...].astype(jnp.float32) + g1 * c1_ref[...].astype(jnp.float32)
    x2 = _ln(h_ref[...] + moe, g_ref[0], b_ref[0])
    x2_ref[...] = x2

    lg = jnp.dot(x2.astype(jnp.bfloat16), wr_ref[...],
                 preferred_element_type=jnp.float32) + br_ref[0]
    lane = jax.lax.broadcasted_iota(jnp.int32, lg.shape, 1)
    l8 = jnp.where(lane < N_EXPERTS, lg, _NEG)
    q0, q1, a0, a1 = _top2(l8)
    oh0 = (lane == a0).astype(jnp.float32)
    oh1 = (lane == a1).astype(jnp.float32)
    accf_ref[...] += jnp.sum(oh0 + oh1, axis=0, keepdims=True)
    accp_ref[...] += jnp.sum(q0 * oh0 + q1 * oh1, axis=0, keepdims=True)

    @pl.when(t == (S // S_TILE) - 1)
    def _():
        f = accf_ref[...] / float(S)
        p = accp_ref[...] / float(S)
        aux = jnp.float32(N_EXPERTS) * jnp.sum(f * p)
        aux_ref[...] = jnp.full((8, 128), aux, jnp.float32)


def _final(h, c0, c1, gates, ln_g, ln_b, Wr_pad, br_r):
    return pl.pallas_call(
        _final_body,
        grid=(S // S_TILE,),
        in_specs=[
            pl.BlockSpec((S_TILE, D_MODEL), lambda t: (t, 0)),
            pl.BlockSpec((S_TILE, D_MODEL), lambda t: (t, 0)),
            pl.BlockSpec((S_TILE, D_MODEL), lambda t: (t, 0)),
            pl.BlockSpec((S_TILE, 128), lambda t: (t, 0)),
            pl.BlockSpec((8, D_MODEL), lambda t: (0, 0)),
            pl.BlockSpec((8, D_MODEL), lambda t: (0, 0)),
            pl.BlockSpec((D_MODEL, 128), lambda t: (0, 0)),
            pl.BlockSpec((8, 128), lambda t: (0, 0)),
        ],
        out_specs=[
            pl.BlockSpec((S_TILE, D_MODEL), lambda t: (t, 0)),
            pl.BlockSpec((8, 128), lambda t: (0, 0)),
        ],
        out_shape=[
            jax.ShapeDtypeStruct((S, D_MODEL), jnp.float32),
            jax.ShapeDtypeStruct((8, 128), jnp.float32),
        ],
        scratch_shapes=[
            pltpu.VMEM((1, 128), jnp.float32),
            pltpu.VMEM((1, 128), jnp.float32),
        ],
    )(h, c0, c1, gates, ln_g, ln_b, Wr_pad, br_r)


def kernel(x, Wq, bq, Wk, bk, Wv, bv, Wo, bo, ln1_g, ln1_b, ln2_g, ln2_b,
           Wr, br, W1, b1, W2, b2):
    x2d = x.reshape(S, D_MODEL)

    Wqkv = jnp.concatenate([
        Wq.reshape(D_MODEL, 8, 128).transpose(1, 0, 2),
        Wk.reshape(D_MODEL, 8, 128).transpose(1, 0, 2),
        Wv.reshape(D_MODEL, 8, 128).transpose(1, 0, 2),
    ], axis=2)
    bqkv = jnp.concatenate([bq.reshape(8, 128), bk.reshape(8, 128),
                            bv.reshape(8, 128)], axis=1).reshape(8, 1, 384)

    Wr_pad = jnp.pad(Wr, ((0, 0), (0, 128 - N_EXPERTS)))
    br_pad = jnp.pad(br.reshape(1, N_EXPERTS),
                     ((0, 0), (0, 128 - N_EXPERTS)))
    br_r = jnp.broadcast_to(br_pad, (8, 128))

    def bcast8(v):
        return jnp.broadcast_to(v.reshape(1, D_MODEL), (8, D_MODEL))

    attn = _attention(x2d.astype(jnp.bfloat16), Wqkv.astype(jnp.bfloat16),
                      bqkv)
    h, h_bf, logits = _postattn(attn, x2d, Wo.astype(jnp.bfloat16),
                                bcast8(bo), bcast8(ln1_g), bcast8(ln1_b),
                                Wr_pad.astype(jnp.bfloat16))
    logits = logits + br_pad

    gates, posmat, temat = _route_meta(logits)
    pos2 = posmat[:, :2].reshape(1, 2 * N_ASSIGN)
    te = temat[:MAX_TILES, 0]

    h2 = jnp.concatenate([h_bf, h_bf], axis=0)
    hg2 = _sc_scatter_rows(_to_words(h2, 2 * N_ASSIGN), pos2, 2 * MAX_ROWS)
    hg = _from_words(hg2, MAX_ROWS)

    W1b = W1.astype(jnp.bfloat16)
    W2b = W2.astype(jnp.bfloat16)
    og = _moe_ffn(te, hg, W1b, b1.reshape(N_EXPERTS, 1, D_FF),
                  W2b, b2.reshape(N_EXPERTS, 1, D_MODEL))

    comb = _sc_gather_rows(_to_words(og, 2 * MAX_ROWS), pos2)
    comb = _from_words(comb, N_ASSIGN)
    c0 = comb[:S]
    c1 = comb[S:]

    x2, auxm = _final(h, c0, c1, gates, bcast8(ln2_g), bcast8(ln2_b),
                      Wr_pad.astype(jnp.bfloat16), br_r)
    aux = auxm[0, 0]
    return (x2.reshape(1, S, D_MODEL), aux)


# speedup vs baseline: 11.8905x; 11.8905x over previous
import jax
import jax.numpy as jnp
from jax.experimental import pallas as pl
from jax.experimental.pallas import tpu as pltpu

D_MODEL = 1024
N_HEADS = 16
D_FF = 4096
N_EXPERTS = 8
S = 2048
N_ASSIGN = 2 * S
ROW_T = 256
MAX_TILES = 24
MAX_ROWS = MAX_TILES * ROW_T
Q_TILE = 512
S_TILE = 256


def _attn_body(xq_ref, xf_ref, w_ref, b_ref, o_ref, kv_ref):
    qt = pl.program_id(1)
    w = w_ref[0]
    b = b_ref[0, 0]

    @pl.when(qt == 0)
    def _():
        xf = xf_ref[...]
        kv_ref[:, :128] = (jnp.dot(xf, w[:, 128:256],
                                   preferred_element_type=jnp.float32)
                           + b[128:256]).astype(jnp.bfloat16)
        kv_ref[:, 128:] = (jnp.dot(xf, w[:, 256:384],
                                   preferred_element_type=jnp.float32)
                           + b[256:384]).astype(jnp.bfloat16)

    xq = xq_ref[...]
    q2 = (jnp.dot(xq, w[:, :128], preferred_element_type=jnp.float32)
          + b[:128]).astype(jnp.bfloat16)
    k2 = kv_ref[:, :128]
    v2 = kv_ref[:, 128:]
    outs = []
    for h in range(2):
        q = q2[:, 64 * h:64 * (h + 1)]
        k = k2[:, 64 * h:64 * (h + 1)]
        v = v2[:, 64 * h:64 * (h + 1)]
        s = jax.lax.dot_general(q, k, (((1,), (1,)), ((), ())),
                                preferred_element_type=jnp.float32) * 0.125
        m = jnp.max(s, axis=1, keepdims=True)
        e = jnp.exp(s - m)
        aw = (e / jnp.sum(e, axis=1, keepdims=True)).astype(jnp.bfloat16)
        outs.append(jnp.dot(aw, v, preferred_element_type=jnp.float32))
    o_ref[...] = jnp.concatenate(outs, axis=1)


def _attention(x2d, Wqkv, bqkv):
    return pl.pallas_call(
        _attn_body,
        grid=(N_HEADS // 2, S // Q_TILE),
        in_specs=[
            pl.BlockSpec((Q_TILE, D_MODEL), lambda p, q: (q, 0)),
            pl.BlockSpec((S, D_MODEL), lambda p, q: (0, 0)),
            pl.BlockSpec((1, D_MODEL, 384), lambda p, q: (p, 0, 0)),
            pl.BlockSpec((1, 1, 384), lambda p, q: (p, 0, 0)),
        ],
        out_specs=pl.BlockSpec((Q_TILE, 128), lambda p, q: (q, p)),
        out_shape=jax.ShapeDtypeStruct((S, D_MODEL), jnp.float32),
        scratch_shapes=[pltpu.VMEM((S, 256), jnp.bfloat16)],
    )(x2d, x2d, Wqkv, bqkv)


def _ln(v, g, b):
    mu = jnp.mean(v, axis=-1, keepdims=True)
    var = jnp.mean((v - mu) ** 2, axis=-1, keepdims=True)
    return (v - mu) * jax.lax.rsqrt(var + 1e-5) * g + b


def _postattn_body(a_ref, x_ref, wo_ref, bo_ref, g_ref, b_ref, wr_ref,
                   h_ref, hb_ref, lg_ref):
    a = a_ref[...].astype(jnp.bfloat16)
    proj = jnp.dot(a, wo_ref[...], preferred_element_type=jnp.float32)
    h = _ln(x_ref[...] + proj + bo_ref[0], g_ref[0], b_ref[0])
    h_ref[...] = h
    hb = h.astype(jnp.bfloat16)
    hb_ref[...] = hb
    lg_ref[...] = jnp.dot(hb, wr_ref[...], preferred_element_type=jnp.float32)


def _postattn(attn, x2d, Wo, bo_r, ln_g, ln_b, Wr_pad):
    return pl.pallas_call(
        _postattn_body,
        grid=(S // S_TILE,),
        in_specs=[
            pl.BlockSpec((S_TILE, D_MODEL), lambda t: (t, 0)),
            pl.BlockSpec((S_TILE, D_MODEL), lambda t: (t, 0)),
            pl.BlockSpec((D_MODEL, D_MODEL), lambda t: (0, 0)),
            pl.BlockSpec((8, D_MODEL), lambda t: (0, 0)),
            pl.BlockSpec((8, D_MODEL), lambda t: (0, 0)),
            pl.BlockSpec((8, D_MODEL), lambda t: (0, 0)),
            pl.BlockSpec((D_MODEL, 128), lambda t: (0, 0)),
        ],
        out_specs=[
            pl.BlockSpec((S_TILE, D_MODEL), lambda t: (t, 0)),
            pl.BlockSpec((S_TILE, D_MODEL), lambda t: (t, 0)),
            pl.BlockSpec((S_TILE, 128), lambda t: (t, 0)),
        ],
        out_shape=[
            jax.ShapeDtypeStruct((S, D_MODEL), jnp.float32),
            jax.ShapeDtypeStruct((S, D_MODEL), jnp.bfloat16),
            jax.ShapeDtypeStruct((S, 128), jnp.float32),
        ],
    )(attn, x2d, Wo, bo_r, ln_g, ln_b, Wr_pad)


def _top2(l8):
    lane = jax.lax.broadcasted_iota(jnp.int32, l8.shape, 1)
    m = jnp.max(l8, axis=1, keepdims=True)
    p = jnp.exp(l8 - m)
    m1 = jnp.max(p, axis=1, keepdims=True)
    a0 = jnp.min(jnp.where(p == m1, lane, N_EXPERTS), axis=1, keepdims=True)
    p2 = jnp.where(lane == a0, -1.0, p)
    m2 = jnp.max(p2, axis=1, keepdims=True)
    a1 = jnp.min(jnp.where(p2 == m2, lane, N_EXPERTS), axis=1, keepdims=True)
    tot = m1 + m2
    return m1 / tot, m2 / tot, a0, a1


_NEG = -1e30


def _route_meta_body(lg_ref, gates_ref, pos_ref, te_ref, e_scr, rank_scr):
    lane = jax.lax.broadcasted_iota(jnp.int32, (S, 128), 1)
    l8 = jnp.where(lane < N_EXPERTS, lg_ref[...], _NEG)
    g0, g1, a0, a1 = _top2(l8)
    gates_ref[...] = jnp.where(lane == 0, g0, 0.0) + jnp.where(lane == 1, g1, 0.0)

    e_col = jnp.concatenate([a0, a1], axis=0)
    lane4 = jax.lax.broadcasted_iota(jnp.int32, (N_ASSIGN, 128), 1)
    E = (lane4 == e_col).astype(jnp.float32)
    e_scr[...] = E

    r = jax.lax.broadcasted_iota(jnp.int32, (128, 128), 0)
    c = jax.lax.broadcasted_iota(jnp.int32, (128, 128), 1)
    T128 = (r >= c).astype(jnp.float32)

    def chunk1(i, _):
        Ec = e_scr[pl.ds(i * 128, 128), :]
        C = jnp.dot(T128, Ec, preferred_element_type=jnp.float32)
        rank_scr[pl.ds(i * 128, 128), :] = C - Ec
        return 0

    jax.lax.fori_loop(0, N_ASSIGN // 128, chunk1, 0)

    nch = N_ASSIGN // 128
    Mrow = jax.lax.broadcasted_iota(jnp.int32, (nch, N_ASSIGN), 0)
    Mcol = jax.lax.broadcasted_iota(jnp.int32, (nch, N_ASSIGN), 1) // 128
    M = (Mrow == Mcol).astype(jnp.float32)
    Sm = jnp.dot(M, E, preferred_element_type=jnp.float32)

    r32 = jax.lax.broadcasted_iota(jnp.int32, (nch, nch), 0)
    c32 = jax.lax.broadcasted_iota(jnp.int32, (nch, nch), 1)
    T32s = (r32 > c32).astype(jnp.float32)
    O = jnp.dot(T32s, Sm, preferred_element_type=jnp.float32)
    MTrow = jax.lax.broadcasted_iota(jnp.int32, (N_ASSIGN, nch), 0) // 128
    MTcol = jax.lax.broadcasted_iota(jnp.int32, (N_ASSIGN, nch), 1)
    MT = (MTrow == MTcol).astype(jnp.float32)
    Oh = jnp.floor(O * (1.0 / 256.0))
    Ol = O - 256.0 * Oh
    OB = (256.0 * jnp.dot(MT, Oh, preferred_element_type=jnp.float32)
          + jnp.dot(MT, Ol, preferred_element_type=jnp.float32))

    tot = jnp.sum(Sm, axis=0, keepdims=True)
    pc = jnp.ceil(tot / ROW_T) * ROW_T
    cu = (jax.lax.broadcasted_iota(jnp.int32, (128, 128), 0) <
          jax.lax.broadcasted_iota(jnp.int32, (128, 128), 1)).astype(jnp.float32)
    pc8 = jnp.broadcast_to(pc * (1.0 / ROW_T), (8, 128))
    off = ROW_T * jnp.dot(pc8, cu, preferred_element_type=jnp.float32)[0:1, :]

    posm = rank_scr[...] + OB + off
    pos_col = jnp.sum(posm * E, axis=1, keepdims=True)
    pos_ref[...] = jnp.broadcast_to(pos_col, (N_ASSIGN, 128)).astype(jnp.int32)

    end = off + pc
    t_col = (jax.lax.broadcasted_iota(jnp.int32, (32, 128), 0) *
             ROW_T).astype(jnp.float32)
    lane32 = jax.lax.broadcasted_iota(jnp.int32, (32, 128), 1)
    fin = jnp.where((t_col >= end) & (lane32 < N_EXPERTS), 1.0, 0.0)
    te = jnp.minimum(jnp.sum(fin, axis=1, keepdims=True),
                     float(N_EXPERTS - 1))
    te_ref[...] = jnp.broadcast_to(te, (32, 128)).astype(jnp.int32)


def _route_meta(logits):
    return pl.pallas_call(
        _route_meta_body,
        grid=(1,),
        in_specs=[pl.BlockSpec((S, 128), lambda i: (0, 0))],
        out_specs=[
            pl.BlockSpec((S, 128), lambda i: (0, 0)),
            pl.BlockSpec((N_ASSIGN, 128), lambda i: (0, 0)),
            pl.BlockSpec((32, 128), lambda i: (0, 0)),
        ],
        out_shape=[
            jax.ShapeDtypeStruct((S, 128), jnp.float32),
            jax.ShapeDtypeStruct((N_ASSIGN, 128), jnp.int32),
            jax.ShapeDtypeStruct((32, 128), jnp.int32),
        ],
        scratch_shapes=[
            pltpu.VMEM((N_ASSIGN, 128), jnp.float32),
            pltpu.VMEM((N_ASSIGN, 128), jnp.float32),
        ],
    )(logits)


def _ffn_body(te_ref, pos_ref, hb_ref, w1_ref, b1_ref, w2_ref, b2_ref,
              og_ref):
    t = pl.program_id(0)
    pos_col = pos_ref[:, 0:1]
    r_row = jax.lax.broadcasted_iota(jnp.int32, (1, ROW_T), 1) + t * ROW_T
    PT = (pos_col == r_row).astype(jnp.bfloat16)
    hb = hb_ref[...]
    dn = (((0,), (0,)), ((), ()))
    hg = (jax.lax.dot_general(PT[:S], hb, dn,
                              preferred_element_type=jnp.float32)
          + jax.lax.dot_general(PT[S:], hb, dn,
                                preferred_element_type=jnp.float32))
    hg = hg.astype(jnp.bfloat16)
    h1 = jnp.dot(hg, w1_ref[0], preferred_element_type=jnp.float32)
    h1 = jnp.maximum(h1 + b1_ref[0, 0], 0.0).astype(jnp.bfloat16)
    og = jnp.dot(h1, w2_ref[0], preferred_element_type=jnp.float32)
    og_ref[...] = (og + b2_ref[0, 0]).astype(jnp.bfloat16)


def _moe_ffn(te, posmat, h_bf, W1b, b1r, W2b, b2r):
    grid_spec = pltpu.PrefetchScalarGridSpec(
        num_scalar_prefetch=1,
        grid=(MAX_TILES,),
        in_specs=[
            pl.BlockSpec((N_ASSIGN, 128), lambda t, te: (0, 0)),
            pl.BlockSpec((S, D_MODEL), lambda t, te: (0, 0)),
            pl.BlockSpec((1, D_MODEL, D_FF), lambda t, te: (te[t], 0, 0)),
            pl.BlockSpec((1, 1, D_FF), lambda t, te: (te[t], 0, 0)),
            pl.BlockSpec((1, D_FF, D_MODEL), lambda t, te: (te[t], 0, 0)),
            pl.BlockSpec((1, 1, D_MODEL), lambda t, te: (te[t], 0, 0)),
        ],
        out_specs=pl.BlockSpec((ROW_T, D_MODEL), lambda t, te: (t, 0)),
    )
    return pl.pallas_call(
        _ffn_body,
        grid_spec=grid_spec,
        out_shape=jax.ShapeDtypeStruct((MAX_ROWS, D_MODEL), jnp.bfloat16),
    )(te, posmat, h_bf, W1b, b1r, W2b, b2r)


def _final_body(h_ref, og_ref, p0_ref, p1_ref, gt_ref, g_ref, b_ref, wr_ref,
                br_ref, x2_ref, aux_ref, accf_ref, accp_ref):
    t = pl.program_id(0)

    @pl.when(t == 0)
    def _():
        accf_ref[...] = jnp.zeros_like(accf_ref)
        accp_ref[...] = jnp.zeros_like(accp_ref)

    lane6 = jax.lax.broadcasted_iota(jnp.int32, (S_TILE, MAX_ROWS), 1)
    og = og_ref[...]
    C0 = (lane6 == p0_ref[:, 0:1]).astype(jnp.bfloat16)
    C1 = (lane6 == p1_ref[:, 0:1]).astype(jnp.bfloat16)
    c0 = jnp.dot(C0, og, preferred_element_type=jnp.float32)
    c1 = jnp.dot(C1, og, preferred_element_type=jnp.float32)

    g0 = gt_ref[:, 0:1].astype(jnp.bfloat16).astype(jnp.float32)
    g1 = gt_ref[:, 1:2].astype(jnp.bfloat16).astype(jnp.float32)
    moe = g0 * c0 + g1 * c1
    x2 = _ln(h_ref[...] + moe, g_ref[0], b_ref[0])
    x2_ref[...] = x2

    lg = jnp.dot(x2.astype(jnp.bfloat16), wr_ref[...],
                 preferred_element_type=jnp.float32) + br_ref[0]
    lane = jax.lax.broadcasted_iota(jnp.int32, lg.shape, 1)
    l8 = jnp.where(lane < N_EXPERTS, lg, _NEG)
    q0, q1, a0, a1 = _top2(l8)
    oh0 = (lane == a0).astype(jnp.float32)
    oh1 = (lane == a1).astype(jnp.float32)
    accf_ref[...] += jnp.sum(oh0 + oh1, axis=0, keepdims=True)
    accp_ref[...] += jnp.sum(q0 * oh0 + q1 * oh1, axis=0, keepdims=True)

    @pl.when(t == (S // S_TILE) - 1)
    def _():
        f = accf_ref[...] / float(S)
        p = accp_ref[...] / float(S)
        aux = jnp.float32(N_EXPERTS) * jnp.sum(f * p)
        aux_ref[...] = jnp.full((8, 128), aux, jnp.float32)


def _final(h, og, posmat, gates, ln_g, ln_b, Wr_pad, br_r):
    return pl.pallas_call(
        _final_body,
        grid=(S // S_TILE,),
        in_specs=[
            pl.BlockSpec((S_TILE, D_MODEL), lambda t: (t, 0)),
            pl.BlockSpec((MAX_ROWS, D_MODEL), lambda t: (0, 0)),
            pl.BlockSpec((S_TILE, 128), lambda t: (t, 0)),
            pl.BlockSpec((S_TILE, 128), lambda t: (t + S // S_TILE, 0)),
            pl.BlockSpec((S_TILE, 128), lambda t: (t, 0)),
            pl.BlockSpec((8, D_MODEL), lambda t: (0, 0)),
            pl.BlockSpec((8, D_MODEL), lambda t: (0, 0)),
            pl.BlockSpec((D_MODEL, 128), lambda t: (0, 0)),
            pl.BlockSpec((8, 128), lambda t: (0, 0)),
        ],
        out_specs=[
            pl.BlockSpec((S_TILE, D_MODEL), lambda t: (t, 0)),
            pl.BlockSpec((8, 128), lambda t: (0, 0)),
        ],
        out_shape=[
            jax.ShapeDtypeStruct((S, D_MODEL), jnp.float32),
            jax.ShapeDtypeStruct((8, 128), jnp.float32),
        ],
        scratch_shapes=[
            pltpu.VMEM((1, 128), jnp.float32),
            pltpu.VMEM((1, 128), jnp.float32),
        ],
    )(h, og, posmat, posmat, gates, ln_g, ln_b, Wr_pad, br_r)


def kernel(x, Wq, bq, Wk, bk, Wv, bv, Wo, bo, ln1_g, ln1_b, ln2_g, ln2_b,
           Wr, br, W1, b1, W2, b2):
    x2d = x.reshape(S, D_MODEL)

    Wqkv = jnp.concatenate([
        Wq.reshape(D_MODEL, 8, 128).transpose(1, 0, 2),
        Wk.reshape(D_MODEL, 8, 128).transpose(1, 0, 2),
        Wv.reshape(D_MODEL, 8, 128).transpose(1, 0, 2),
    ], axis=2)
    bqkv = jnp.concatenate([bq.reshape(8, 128), bk.reshape(8, 128),
                            bv.reshape(8, 128)], axis=1).reshape(8, 1, 384)

    Wr_pad = jnp.pad(Wr, ((0, 0), (0, 128 - N_EXPERTS)))
    br_pad = jnp.pad(br.reshape(1, N_EXPERTS),
                     ((0, 0), (0, 128 - N_EXPERTS)))
    br_r = jnp.broadcast_to(br_pad, (8, 128))

    def bcast8(v):
        return jnp.broadcast_to(v.reshape(1, D_MODEL), (8, D_MODEL))

    attn = _attention(x2d.astype(jnp.bfloat16), Wqkv.astype(jnp.bfloat16),
                      bqkv)
    h, h_bf, logits = _postattn(attn, x2d, Wo.astype(jnp.bfloat16),
                                bcast8(bo), bcast8(ln1_g), bcast8(ln1_b),
                                Wr_pad.astype(jnp.bfloat16))
    logits = logits + br_pad

    gates, posmat, temat = _route_meta(logits)
    te = temat[:MAX_TILES, 0]

    W1b = W1.astype(jnp.bfloat16)
    W2b = W2.astype(jnp.bfloat16)
    og = _moe_ffn(te, posmat, h_bf, W1b, b1.reshape(N_EXPERTS, 1, D_FF),
                  W2b, b2.reshape(N_EXPERTS, 1, D_MODEL))

    x2, auxm = _final(h, og, posmat, gates, bcast8(ln2_g), bcast8(ln2_b),
                      Wr_pad.astype(jnp.bfloat16), br_r)
    aux = auxm[0, 0]
    return (x2.reshape(1, S, D_MODEL), aux)


# speedup vs baseline: 12.2167x; 1.0274x over previous
import jax
import jax.numpy as jnp
from jax.experimental import pallas as pl
from jax.experimental.pallas import tpu as pltpu

D_MODEL = 1024
N_HEADS = 16
D_FF = 4096
N_EXPERTS = 8
S = 2048
N_ASSIGN = 2 * S
ROW_T = 256
MAX_TILES = 24
MAX_ROWS = MAX_TILES * ROW_T
Q_TILE = 512
S_TILE = 256


def _attn_body(xq_ref, xf_ref, wq_ref, wk_ref, wv_ref, bq_ref, bk_ref,
               bv_ref, o_ref, kv_ref):
    qt = pl.program_id(1)

    @pl.when(qt == 0)
    def _():
        xf = xf_ref[...]
        kv_ref[:, :128] = (jnp.dot(xf, wk_ref[...],
                                   preferred_element_type=jnp.float32)
                           + bk_ref[0, 0]).astype(jnp.bfloat16)
        kv_ref[:, 128:] = (jnp.dot(xf, wv_ref[...],
                                   preferred_element_type=jnp.float32)
                           + bv_ref[0, 0]).astype(jnp.bfloat16)

    xq = xq_ref[...]
    q2 = (jnp.dot(xq, wq_ref[...], preferred_element_type=jnp.float32)
          + bq_ref[0, 0]).astype(jnp.bfloat16)
    k2 = kv_ref[:, :128]
    v2 = kv_ref[:, 128:]
    outs = []
    for h in range(2):
        q = q2[:, 64 * h:64 * (h + 1)]
        k = k2[:, 64 * h:64 * (h + 1)]
        v = v2[:, 64 * h:64 * (h + 1)]
        s = jax.lax.dot_general(q, k, (((1,), (1,)), ((), ())),
                                preferred_element_type=jnp.float32) * 0.125
        m = jnp.max(s, axis=1, keepdims=True)
        e = jnp.exp(s - m)
        aw = (e / jnp.sum(e, axis=1, keepdims=True)).astype(jnp.bfloat16)
        outs.append(jnp.dot(aw, v, preferred_element_type=jnp.float32))
    o_ref[...] = jnp.concatenate(outs, axis=1)


def _attention(x2d, Wq, Wk, Wv, bq, bk, bv):
    wspec = pl.BlockSpec((D_MODEL, 128), lambda p, q: (0, p))
    bspec = pl.BlockSpec((1, 1, 128), lambda p, q: (p, 0, 0))
    return pl.pallas_call(
        _attn_body,
        grid=(N_HEADS // 2, S // Q_TILE),
        in_specs=[
            pl.BlockSpec((Q_TILE, D_MODEL), lambda p, q: (q, 0)),
            pl.BlockSpec((S, D_MODEL), lambda p, q: (0, 0)),
            wspec, wspec, wspec, bspec, bspec, bspec,
        ],
        out_specs=pl.BlockSpec((Q_TILE, 128), lambda p, q: (q, p)),
        out_shape=jax.ShapeDtypeStruct((S, D_MODEL), jnp.float32),
        scratch_shapes=[pltpu.VMEM((S, 256), jnp.bfloat16)],
    )(x2d, x2d, Wq.astype(jnp.bfloat16), Wk.astype(jnp.bfloat16),
      Wv.astype(jnp.bfloat16), bq.reshape(8, 1, 128), bk.reshape(8, 1, 128),
      bv.reshape(8, 1, 128))


def _ln(v, g, b):
    mu = jnp.mean(v, axis=-1, keepdims=True)
    var = jnp.mean((v - mu) ** 2, axis=-1, keepdims=True)
    return (v - mu) * jax.lax.rsqrt(var + 1e-5) * g + b


def _postattn_body(a_ref, x_ref, wo_ref, bo_ref, g_ref, b_ref, wr_ref,
                   h_ref, hb_ref, lg_ref):
    a = a_ref[...].astype(jnp.bfloat16)
    proj = jnp.dot(a, wo_ref[...], preferred_element_type=jnp.float32)
    h = _ln(x_ref[...] + proj + bo_ref[0], g_ref[0], b_ref[0])
    h_ref[...] = h
    hb = h.astype(jnp.bfloat16)
    hb_ref[...] = hb
    lg_ref[...] = jnp.dot(hb, wr_ref[...], preferred_element_type=jnp.float32)


def _postattn(attn, x2d, Wo, bo_r, ln_g, ln_b, Wr_pad):
    return pl.pallas_call(
        _postattn_body,
        grid=(S // S_TILE,),
        in_specs=[
            pl.BlockSpec((S_TILE, D_MODEL), lambda t: (t, 0)),
            pl.BlockSpec((S_TILE, D_MODEL), lambda t: (t, 0)),
            pl.BlockSpec((D_MODEL, D_MODEL), lambda t: (0, 0)),
            pl.BlockSpec((8, D_MODEL), lambda t: (0, 0)),
            pl.BlockSpec((8, D_MODEL), lambda t: (0, 0)),
            pl.BlockSpec((8, D_MODEL), lambda t: (0, 0)),
            pl.BlockSpec((D_MODEL, 128), lambda t: (0, 0)),
        ],
        out_specs=[
            pl.BlockSpec((S_TILE, D_MODEL), lambda t: (t, 0)),
            pl.BlockSpec((S_TILE, D_MODEL), lambda t: (t, 0)),
            pl.BlockSpec((S_TILE, 128), lambda t: (t, 0)),
        ],
        out_shape=[
            jax.ShapeDtypeStruct((S, D_MODEL), jnp.float32),
            jax.ShapeDtypeStruct((S, D_MODEL), jnp.bfloat16),
            jax.ShapeDtypeStruct((S, 128), jnp.float32),
        ],
    )(attn, x2d, Wo, bo_r, ln_g, ln_b, Wr_pad)


def _top2(l8):
    lane = jax.lax.broadcasted_iota(jnp.int32, l8.shape, 1)
    m = jnp.max(l8, axis=1, keepdims=True)
    p = jnp.exp(l8 - m)
    m1 = jnp.max(p, axis=1, keepdims=True)
    a0 = jnp.min(jnp.where(p == m1, lane, N_EXPERTS), axis=1, keepdims=True)
    p2 = jnp.where(lane == a0, -1.0, p)
    m2 = jnp.max(p2, axis=1, keepdims=True)
    a1 = jnp.min(jnp.where(p2 == m2, lane, N_EXPERTS), axis=1, keepdims=True)
    tot = m1 + m2
    return m1 / tot, m2 / tot, a0, a1


_NEG = -1e30


def _route_meta_body(lg_ref, gates_ref, pos_ref, te_ref, e_scr, rank_scr):
    lane = jax.lax.broadcasted_iota(jnp.int32, (S, 128), 1)
    l8 = jnp.where(lane < N_EXPERTS, lg_ref[...], _NEG)
    g0, g1, a0, a1 = _top2(l8)
    gates_ref[...] = jnp.where(lane == 0, g0, 0.0) + jnp.where(lane == 1, g1, 0.0)

    e_col = jnp.concatenate([a0, a1], axis=0)
    lane4 = jax.lax.broadcasted_iota(jnp.int32, (N_ASSIGN, 128), 1)
    E = (lane4 == e_col).astype(jnp.float32)
    e_scr[...] = E

    r = jax.lax.broadcasted_iota(jnp.int32, (128, 128), 0)
    c = jax.lax.broadcasted_iota(jnp.int32, (128, 128), 1)
    T128 = (r >= c).astype(jnp.float32)

    def chunk1(i, _):
        Ec = e_scr[pl.ds(i * 128, 128), :]
        C = jnp.dot(T128, Ec, preferred_element_type=jnp.float32)
        rank_scr[pl.ds(i * 128, 128), :] = C - Ec
        return 0

    jax.lax.fori_loop(0, N_ASSIGN // 128, chunk1, 0)

    nch = N_ASSIGN // 128
    Mrow = jax.lax.broadcasted_iota(jnp.int32, (nch, N_ASSIGN), 0)
    Mcol = jax.lax.broadcasted_iota(jnp.int32, (nch, N_ASSIGN), 1) // 128
    M = (Mrow == Mcol).astype(jnp.float32)
    Sm = jnp.dot(M, E, preferred_element_type=jnp.float32)

    r32 = jax.lax.broadcasted_iota(jnp.int32, (nch, nch), 0)
    c32 = jax.lax.broadcasted_iota(jnp.int32, (nch, nch), 1)
    T32s = (r32 > c32).astype(jnp.float32)
    O = jnp.dot(T32s, Sm, preferred_element_type=jnp.float32)
    MTrow = jax.lax.broadcasted_iota(jnp.int32, (N_ASSIGN, nch), 0) // 128
    MTcol = jax.lax.broadcasted_iota(jnp.int32, (N_ASSIGN, nch), 1)
    MT = (MTrow == MTcol).astype(jnp.float32)
    Oh = jnp.floor(O * (1.0 / 256.0))
    Ol = O - 256.0 * Oh
    OB = (256.0 * jnp.dot(MT, Oh, preferred_element_type=jnp.float32)
          + jnp.dot(MT, Ol, preferred_element_type=jnp.float32))

    tot = jnp.sum(Sm, axis=0, keepdims=True)
    pc = jnp.ceil(tot / ROW_T) * ROW_T
    cu = (jax.lax.broadcasted_iota(jnp.int32, (128, 128), 0) <
          jax.lax.broadcasted_iota(jnp.int32, (128, 128), 1)).astype(jnp.float32)
    pc8 = jnp.broadcast_to(pc * (1.0 / ROW_T), (8, 128))
    off = ROW_T * jnp.dot(pc8, cu, preferred_element_type=jnp.float32)[0:1, :]

    posm = rank_scr[...] + OB + off
    pos_col = jnp.sum(posm * E, axis=1, keepdims=True)
    pos_ref[...] = jnp.broadcast_to(pos_col, (N_ASSIGN, 128)).astype(jnp.int32)

    end = off + pc
    t_col = (jax.lax.broadcasted_iota(jnp.int32, (32, 128), 0) *
             ROW_T).astype(jnp.float32)
    lane32 = jax.lax.broadcasted_iota(jnp.int32, (32, 128), 1)
    fin = jnp.where((t_col >= end) & (lane32 < N_EXPERTS), 1.0, 0.0)
    te = jnp.minimum(jnp.sum(fin, axis=1, keepdims=True),
                     float(N_EXPERTS - 1))
    te_ref[...] = jnp.broadcast_to(te, (32, 128)).astype(jnp.int32)


def _route_meta(logits):
    return pl.pallas_call(
        _route_meta_body,
        grid=(1,),
        in_specs=[pl.BlockSpec((S, 128), lambda i: (0, 0))],
        out_specs=[
            pl.BlockSpec((S, 128), lambda i: (0, 0)),
            pl.BlockSpec((N_ASSIGN, 128), lambda i: (0, 0)),
            pl.BlockSpec((32, 128), lambda i: (0, 0)),
        ],
        out_shape=[
            jax.ShapeDtypeStruct((S, 128), jnp.float32),
            jax.ShapeDtypeStruct((N_ASSIGN, 128), jnp.int32),
            jax.ShapeDtypeStruct((32, 128), jnp.int32),
        ],
        scratch_shapes=[
            pltpu.VMEM((N_ASSIGN, 128), jnp.float32),
            pltpu.VMEM((N_ASSIGN, 128), jnp.float32),
        ],
    )(logits)


def _ffn_body(te_ref, pos_ref, hb_ref, w1_ref, b1_ref, w2_ref, b2_ref,
              og_ref):
    t = pl.program_id(0)
    pos_col = pos_ref[:, 0:1]
    r_row = jax.lax.broadcasted_iota(jnp.int32, (1, ROW_T), 1) + t * ROW_T
    PT = (pos_col == r_row).astype(jnp.bfloat16)
    hb = hb_ref[...]
    dn = (((0,), (0,)), ((), ()))
    hg = (jax.lax.dot_general(PT[:S], hb, dn,
                              preferred_element_type=jnp.float32)
          + jax.lax.dot_general(PT[S:], hb, dn,
                                preferred_element_type=jnp.float32))
    hg = hg.astype(jnp.bfloat16)
    h1 = jnp.dot(hg, w1_ref[0], preferred_element_type=jnp.float32)
    h1 = jnp.maximum(h1 + b1_ref[0, 0], 0.0).astype(jnp.bfloat16)
    og = jnp.dot(h1, w2_ref[0], preferred_element_type=jnp.float32)
    og_ref[...] = (og + b2_ref[0, 0]).astype(jnp.bfloat16)


def _moe_ffn(te, posmat, h_bf, W1b, b1r, W2b, b2r):
    grid_spec = pltpu.PrefetchScalarGridSpec(
        num_scalar_prefetch=1,
        grid=(MAX_TILES,),
        in_specs=[
            pl.BlockSpec((N_ASSIGN, 128), lambda t, te: (0, 0)),
            pl.BlockSpec((S, D_MODEL), lambda t, te: (0, 0)),
            pl.BlockSpec((1, D_MODEL, D_FF), lambda t, te: (te[t], 0, 0)),
            pl.BlockSpec((1, 1, D_FF), lambda t, te: (te[t], 0, 0)),
            pl.BlockSpec((1, D_FF, D_MODEL), lambda t, te: (te[t], 0, 0)),
            pl.BlockSpec((1, 1, D_MODEL), lambda t, te: (te[t], 0, 0)),
        ],
        out_specs=pl.BlockSpec((ROW_T, D_MODEL), lambda t, te: (t, 0)),
    )
    return pl.pallas_call(
        _ffn_body,
        grid_spec=grid_spec,
        out_shape=jax.ShapeDtypeStruct((MAX_ROWS, D_MODEL), jnp.bfloat16),
    )(te, posmat, h_bf, W1b, b1r, W2b, b2r)


def _final_body(h_ref, og_ref, p0_ref, p1_ref, gt_ref, g_ref, b_ref, wr_ref,
                br_ref, x2_ref, aux_ref, accf_ref, accp_ref):
    t = pl.program_id(0)

    @pl.when(t == 0)
    def _():
        accf_ref[...] = jnp.zeros_like(accf_ref)
        accp_ref[...] = jnp.zeros_like(accp_ref)

    lane6 = jax.lax.broadcasted_iota(jnp.int32, (S_TILE, MAX_ROWS), 1)
    og = og_ref[...]
    C0 = (lane6 == p0_ref[:, 0:1]).astype(jnp.bfloat16)
    C1 = (lane6 == p1_ref[:, 0:1]).astype(jnp.bfloat16)
    c0 = jnp.dot(C0, og, preferred_element_type=jnp.float32)
    c1 = jnp.dot(C1, og, preferred_element_type=jnp.float32)

    g0 = gt_ref[:, 0:1].astype(jnp.bfloat16).astype(jnp.float32)
    g1 = gt_ref[:, 1:2].astype(jnp.bfloat16).astype(jnp.float32)
    moe = g0 * c0 + g1 * c1
    x2 = _ln(h_ref[...] + moe, g_ref[0], b_ref[0])
    x2_ref[...] = x2

    lg = jnp.dot(x2.astype(jnp.bfloat16), wr_ref[...],
                 preferred_element_type=jnp.float32) + br_ref[0]
    lane = jax.lax.broadcasted_iota(jnp.int32, lg.shape, 1)
    l8 = jnp.where(lane < N_EXPERTS, lg, _NEG)
    q0, q1, a0, a1 = _top2(l8)
    oh0 = (lane == a0).astype(jnp.float32)
    oh1 = (lane == a1).astype(jnp.float32)
    accf_ref[...] += jnp.sum(oh0 + oh1, axis=0, keepdims=True)
    accp_ref[...] += jnp.sum(q0 * oh0 + q1 * oh1, axis=0, keepdims=True)

    @pl.when(t == (S // S_TILE) - 1)
    def _():
        f = accf_ref[...] / float(S)
        p = accp_ref[...] / float(S)
        aux = jnp.float32(N_EXPERTS) * jnp.sum(f * p)
        aux_ref[...] = jnp.full((8, 128), aux, jnp.float32)


def _final(h, og, posmat, gates, ln_g, ln_b, Wr_pad, br_r):
    return pl.pallas_call(
        _final_body,
        grid=(S // S_TILE,),
        in_specs=[
            pl.BlockSpec((S_TILE, D_MODEL), lambda t: (t, 0)),
            pl.BlockSpec((MAX_ROWS, D_MODEL), lambda t: (0, 0)),
            pl.BlockSpec((S_TILE, 128), lambda t: (t, 0)),
            pl.BlockSpec((S_TILE, 128), lambda t: (t + S // S_TILE, 0)),
            pl.BlockSpec((S_TILE, 128), lambda t: (t, 0)),
            pl.BlockSpec((8, D_MODEL), lambda t: (0, 0)),
            pl.BlockSpec((8, D_MODEL), lambda t: (0, 0)),
            pl.BlockSpec((D_MODEL, 128), lambda t: (0, 0)),
            pl.BlockSpec((8, 128), lambda t: (0, 0)),
        ],
        out_specs=[
            pl.BlockSpec((S_TILE, D_MODEL), lambda t: (t, 0)),
            pl.BlockSpec((8, 128), lambda t: (0, 0)),
        ],
        out_shape=[
            jax.ShapeDtypeStruct((S, D_MODEL), jnp.float32),
            jax.ShapeDtypeStruct((8, 128), jnp.float32),
        ],
        scratch_shapes=[
            pltpu.VMEM((1, 128), jnp.float32),
            pltpu.VMEM((1, 128), jnp.float32),
        ],
    )(h, og, posmat, posmat, gates, ln_g, ln_b, Wr_pad, br_r)


def kernel(x, Wq, bq, Wk, bk, Wv, bv, Wo, bo, ln1_g, ln1_b, ln2_g, ln2_b,
           Wr, br, W1, b1, W2, b2):
    x2d = x.reshape(S, D_MODEL)

    Wr_pad = jnp.pad(Wr, ((0, 0), (0, 128 - N_EXPERTS)))
    br_pad = jnp.pad(br.reshape(1, N_EXPERTS),
                     ((0, 0), (0, 128 - N_EXPERTS)))
    br_r = jnp.broadcast_to(br_pad, (8, 128))

    def bcast8(v):
        return jnp.broadcast_to(v.reshape(1, D_MODEL), (8, D_MODEL))

    attn = _attention(x2d.astype(jnp.bfloat16), Wq, Wk, Wv, bq, bk, bv)
    h, h_bf, logits = _postattn(attn, x2d, Wo.astype(jnp.bfloat16),
                                bcast8(bo), bcast8(ln1_g), bcast8(ln1_b),
                                Wr_pad.astype(jnp.bfloat16))
    logits = logits + br_pad

    gates, posmat, temat = _route_meta(logits)
    te = temat[:MAX_TILES, 0]

    W1b = W1.astype(jnp.bfloat16)
    W2b = W2.astype(jnp.bfloat16)
    og = _moe_ffn(te, posmat, h_bf, W1b, b1.reshape(N_EXPERTS, 1, D_FF),
                  W2b, b2.reshape(N_EXPERTS, 1, D_MODEL))

    x2, auxm = _final(h, og, posmat, gates, bcast8(ln2_g), bcast8(ln2_b),
                      Wr_pad.astype(jnp.bfloat16), br_r)
    aux = auxm[0, 0]
    return (x2.reshape(1, S, D_MODEL), aux)


# speedup vs baseline: 12.3215x; 1.0086x over previous
import jax
import jax.numpy as jnp
from jax.experimental import pallas as pl
from jax.experimental.pallas import tpu as pltpu

D_MODEL = 1024
N_HEADS = 16
D_FF = 4096
N_EXPERTS = 8
S = 2048
N_ASSIGN = 2 * S
ROW_T = 128
MAX_TILES = 40
MAX_ROWS = MAX_TILES * ROW_T
Q_TILE = 512
S_TILE = 256


def _attn_body(xq_ref, xf_ref, wq_ref, wk_ref, wv_ref, bq_ref, bk_ref,
               bv_ref, o_ref, kv_ref):
    qt = pl.program_id(1)

    @pl.when(qt == 0)
    def _():
        xf = xf_ref[...]
        kv_ref[:, :128] = (jnp.dot(xf, wk_ref[...],
                                   preferred_element_type=jnp.float32)
                           + bk_ref[0, 0]).astype(jnp.bfloat16)
        kv_ref[:, 128:] = (jnp.dot(xf, wv_ref[...],
                                   preferred_element_type=jnp.float32)
                           + bv_ref[0, 0]).astype(jnp.bfloat16)

    xq = xq_ref[...]
    q2 = (jnp.dot(xq, wq_ref[...], preferred_element_type=jnp.float32)
          + bq_ref[0, 0]).astype(jnp.bfloat16)
    k2 = kv_ref[:, :128]
    v2 = kv_ref[:, 128:]
    outs = []
    for h in range(2):
        q = q2[:, 64 * h:64 * (h + 1)]
        k = k2[:, 64 * h:64 * (h + 1)]
        v = v2[:, 64 * h:64 * (h + 1)]
        s = jax.lax.dot_general(q, k, (((1,), (1,)), ((), ())),
                                preferred_element_type=jnp.float32) * 0.125
        m = jnp.max(s, axis=1, keepdims=True)
        e = jnp.exp(s - m)
        aw = (e / jnp.sum(e, axis=1, keepdims=True)).astype(jnp.bfloat16)
        outs.append(jnp.dot(aw, v, preferred_element_type=jnp.float32))
    o_ref[...] = jnp.concatenate(outs, axis=1)


def _attention(x2d, Wq, Wk, Wv, bq, bk, bv):
    wspec = pl.BlockSpec((D_MODEL, 128), lambda p, q: (0, p))
    bspec = pl.BlockSpec((1, 1, 128), lambda p, q: (p, 0, 0))
    return pl.pallas_call(
        _attn_body,
        grid=(N_HEADS // 2, S // Q_TILE),
        in_specs=[
            pl.BlockSpec((Q_TILE, D_MODEL), lambda p, q: (q, 0)),
            pl.BlockSpec((S, D_MODEL), lambda p, q: (0, 0)),
            wspec, wspec, wspec, bspec, bspec, bspec,
        ],
        out_specs=pl.BlockSpec((Q_TILE, 128), lambda p, q: (q, p)),
        out_shape=jax.ShapeDtypeStruct((S, D_MODEL), jnp.float32),
        scratch_shapes=[pltpu.VMEM((S, 256), jnp.bfloat16)],
    )(x2d, x2d, Wq.astype(jnp.bfloat16), Wk.astype(jnp.bfloat16),
      Wv.astype(jnp.bfloat16), bq.reshape(8, 1, 128), bk.reshape(8, 1, 128),
      bv.reshape(8, 1, 128))


def _ln(v, g, b):
    mu = jnp.mean(v, axis=-1, keepdims=True)
    var = jnp.mean((v - mu) ** 2, axis=-1, keepdims=True)
    return (v - mu) * jax.lax.rsqrt(var + 1e-5) * g + b


def _postattn_body(a_ref, x_ref, wo_ref, bo_ref, g_ref, b_ref, wr_ref,
                   h_ref, hb_ref, lg_ref):
    a = a_ref[...].astype(jnp.bfloat16)
    proj = jnp.dot(a, wo_ref[...], preferred_element_type=jnp.float32)
    h = _ln(x_ref[...] + proj + bo_ref[0], g_ref[0], b_ref[0])
    h_ref[...] = h
    hb = h.astype(jnp.bfloat16)
    hb_ref[...] = hb
    lg_ref[...] = jnp.dot(hb, wr_ref[...], preferred_element_type=jnp.float32)


def _postattn(attn, x2d, Wo, bo_r, ln_g, ln_b, Wr_pad):
    return pl.pallas_call(
        _postattn_body,
        grid=(S // S_TILE,),
        in_specs=[
            pl.BlockSpec((S_TILE, D_MODEL), lambda t: (t, 0)),
            pl.BlockSpec((S_TILE, D_MODEL), lambda t: (t, 0)),
            pl.BlockSpec((D_MODEL, D_MODEL), lambda t: (0, 0)),
            pl.BlockSpec((8, D_MODEL), lambda t: (0, 0)),
            pl.BlockSpec((8, D_MODEL), lambda t: (0, 0)),
            pl.BlockSpec((8, D_MODEL), lambda t: (0, 0)),
            pl.BlockSpec((D_MODEL, 128), lambda t: (0, 0)),
        ],
        out_specs=[
            pl.BlockSpec((S_TILE, D_MODEL), lambda t: (t, 0)),
            pl.BlockSpec((S_TILE, D_MODEL), lambda t: (t, 0)),
            pl.BlockSpec((S_TILE, 128), lambda t: (t, 0)),
        ],
        out_shape=[
            jax.ShapeDtypeStruct((S, D_MODEL), jnp.float32),
            jax.ShapeDtypeStruct((S, D_MODEL), jnp.bfloat16),
            jax.ShapeDtypeStruct((S, 128), jnp.float32),
        ],
    )(attn, x2d, Wo, bo_r, ln_g, ln_b, Wr_pad)


def _top2(l8):
    lane = jax.lax.broadcasted_iota(jnp.int32, l8.shape, 1)
    m = jnp.max(l8, axis=1, keepdims=True)
    p = jnp.exp(l8 - m)
    m1 = jnp.max(p, axis=1, keepdims=True)
    a0 = jnp.min(jnp.where(p == m1, lane, N_EXPERTS), axis=1, keepdims=True)
    p2 = jnp.where(lane == a0, -1.0, p)
    m2 = jnp.max(p2, axis=1, keepdims=True)
    a1 = jnp.min(jnp.where(p2 == m2, lane, N_EXPERTS), axis=1, keepdims=True)
    tot = m1 + m2
    return m1 / tot, m2 / tot, a0, a1


_NEG = -1e30


def _route_meta_body(lg_ref, gates_ref, pos_ref, te_ref, e_scr, rank_scr):
    lane = jax.lax.broadcasted_iota(jnp.int32, (S, 128), 1)
    l8 = jnp.where(lane < N_EXPERTS, lg_ref[...], _NEG)
    g0, g1, a0, a1 = _top2(l8)
    gates_ref[...] = jnp.where(lane == 0, g0, 0.0) + jnp.where(lane == 1, g1, 0.0)

    e_col = jnp.concatenate([a0, a1], axis=0)
    lane4 = jax.lax.broadcasted_iota(jnp.int32, (N_ASSIGN, 128), 1)
    E = (lane4 == e_col).astype(jnp.float32)
    e_scr[...] = E

    r = jax.lax.broadcasted_iota(jnp.int32, (128, 128), 0)
    c = jax.lax.broadcasted_iota(jnp.int32, (128, 128), 1)
    T128 = (r >= c).astype(jnp.float32)

    def chunk1(i, _):
        Ec = e_scr[pl.ds(i * 128, 128), :]
        C = jnp.dot(T128, Ec, preferred_element_type=jnp.float32)
        rank_scr[pl.ds(i * 128, 128), :] = C - Ec
        return 0

    jax.lax.fori_loop(0, N_ASSIGN // 128, chunk1, 0)

    nch = N_ASSIGN // 128
    Mrow = jax.lax.broadcasted_iota(jnp.int32, (nch, N_ASSIGN), 0)
    Mcol = jax.lax.broadcasted_iota(jnp.int32, (nch, N_ASSIGN), 1) // 128
    M = (Mrow == Mcol).astype(jnp.float32)
    Sm = jnp.dot(M, E, preferred_element_type=jnp.float32)

    r32 = jax.lax.broadcasted_iota(jnp.int32, (nch, nch), 0)
    c32 = jax.lax.broadcasted_iota(jnp.int32, (nch, nch), 1)
    T32s = (r32 > c32).astype(jnp.float32)
    O = jnp.dot(T32s, Sm, preferred_element_type=jnp.float32)
    MTrow = jax.lax.broadcasted_iota(jnp.int32, (N_ASSIGN, nch), 0) // 128
    MTcol = jax.lax.broadcasted_iota(jnp.int32, (N_ASSIGN, nch), 1)
    MT = (MTrow == MTcol).astype(jnp.float32)
    Oh = jnp.floor(O * (1.0 / 256.0))
    Ol = O - 256.0 * Oh
    OB = (256.0 * jnp.dot(MT, Oh, preferred_element_type=jnp.float32)
          + jnp.dot(MT, Ol, preferred_element_type=jnp.float32))

    tot = jnp.sum(Sm, axis=0, keepdims=True)
    pc = jnp.ceil(tot / ROW_T) * ROW_T
    cu = (jax.lax.broadcasted_iota(jnp.int32, (128, 128), 0) <
          jax.lax.broadcasted_iota(jnp.int32, (128, 128), 1)).astype(jnp.float32)
    pc8 = jnp.broadcast_to(pc * (1.0 / ROW_T), (8, 128))
    off = ROW_T * jnp.dot(pc8, cu, preferred_element_type=jnp.float32)[0:1, :]

    posm = rank_scr[...] + OB + off
    pos_col = jnp.sum(posm * E, axis=1, keepdims=True)
    pos_ref[...] = jnp.broadcast_to(pos_col, (N_ASSIGN, 128)).astype(jnp.int32)

    end = off + pc
    t_col = (jax.lax.broadcasted_iota(jnp.int32, (MAX_TILES, 128), 0) *
             ROW_T).astype(jnp.float32)
    lane32 = jax.lax.broadcasted_iota(jnp.int32, (MAX_TILES, 128), 1)
    fin = jnp.where((t_col >= end) & (lane32 < N_EXPERTS), 1.0, 0.0)
    te = jnp.minimum(jnp.sum(fin, axis=1, keepdims=True),
                     float(N_EXPERTS - 1))
    te_ref[...] = jnp.broadcast_to(te, (MAX_TILES, 128)).astype(jnp.int32)


def _route_meta(logits):
    return pl.pallas_call(
        _route_meta_body,
        grid=(1,),
        in_specs=[pl.BlockSpec((S, 128), lambda i: (0, 0))],
        out_specs=[
            pl.BlockSpec((S, 128), lambda i: (0, 0)),
            pl.BlockSpec((N_ASSIGN, 128), lambda i: (0, 0)),
            pl.BlockSpec((MAX_TILES, 128), lambda i: (0, 0)),
        ],
        out_shape=[
            jax.ShapeDtypeStruct((S, 128), jnp.float32),
            jax.ShapeDtypeStruct((N_ASSIGN, 128), jnp.int32),
            jax.ShapeDtypeStruct((MAX_TILES, 128), jnp.int32),
        ],
        scratch_shapes=[
            pltpu.VMEM((N_ASSIGN, 128), jnp.float32),
            pltpu.VMEM((N_ASSIGN, 128), jnp.float32),
        ],
    )(logits)


def _ffn_body(te_ref, pos_ref, hb_ref, w1_ref, b1_ref, w2_ref, b2_ref,
              og_ref):
    t = pl.program_id(0)
    pos_col = pos_ref[:, 0:1]
    r_row = jax.lax.broadcasted_iota(jnp.int32, (1, ROW_T), 1) + t * ROW_T
    PT = (pos_col == r_row).astype(jnp.bfloat16)
    hb = hb_ref[...]
    dn = (((0,), (0,)), ((), ()))
    hg = (jax.lax.dot_general(PT[:S], hb, dn,
                              preferred_element_type=jnp.float32)
          + jax.lax.dot_general(PT[S:], hb, dn,
                                preferred_element_type=jnp.float32))
    hg = hg.astype(jnp.bfloat16)
    h1 = jnp.dot(hg, w1_ref[0], preferred_element_type=jnp.float32)
    h1 = jnp.maximum(h1 + b1_ref[0, 0], 0.0).astype(jnp.bfloat16)
    og = jnp.dot(h1, w2_ref[0], preferred_element_type=jnp.float32)
    og_ref[...] = (og + b2_ref[0, 0]).astype(jnp.bfloat16)


def _moe_ffn(te, posmat, h_bf, W1b, b1r, W2b, b2r):
    grid_spec = pltpu.PrefetchScalarGridSpec(
        num_scalar_prefetch=1,
        grid=(MAX_TILES,),
        in_specs=[
            pl.BlockSpec((N_ASSIGN, 128), lambda t, te: (0, 0)),
            pl.BlockSpec((S, D_MODEL), lambda t, te: (0, 0)),
            pl.BlockSpec((1, D_MODEL, D_FF), lambda t, te: (te[t], 0, 0)),
            pl.BlockSpec((1, 1, D_FF), lambda t, te: (te[t], 0, 0)),
            pl.BlockSpec((1, D_FF, D_MODEL), lambda t, te: (te[t], 0, 0)),
            pl.BlockSpec((1, 1, D_MODEL), lambda t, te: (te[t], 0, 0)),
        ],
        out_specs=pl.BlockSpec((ROW_T, D_MODEL), lambda t, te: (t, 0)),
    )
    return pl.pallas_call(
        _ffn_body,
        grid_spec=grid_spec,
        out_shape=jax.ShapeDtypeStruct((MAX_ROWS, D_MODEL), jnp.bfloat16),
    )(te, posmat, h_bf, W1b, b1r, W2b, b2r)


def _final_body(h_ref, og_ref, p0_ref, p1_ref, gt_ref, g_ref, b_ref, wr_ref,
                br_ref, x2_ref, aux_ref, accf_ref, accp_ref):
    t = pl.program_id(0)

    @pl.when(t == 0)
    def _():
        accf_ref[...] = jnp.zeros_like(accf_ref)
        accp_ref[...] = jnp.zeros_like(accp_ref)

    lane6 = jax.lax.broadcasted_iota(jnp.int32, (S_TILE, MAX_ROWS), 1)
    og = og_ref[...]
    C0 = (lane6 == p0_ref[:, 0:1]).astype(jnp.bfloat16)
    C1 = (lane6 == p1_ref[:, 0:1]).astype(jnp.bfloat16)
    c0 = jnp.dot(C0, og, preferred_element_type=jnp.float32)
    c1 = jnp.dot(C1, og, preferred_element_type=jnp.float32)

    g0 = gt_ref[:, 0:1].astype(jnp.bfloat16).astype(jnp.float32)
    g1 = gt_ref[:, 1:2].astype(jnp.bfloat16).astype(jnp.float32)
    moe = g0 * c0 + g1 * c1
    x2 = _ln(h_ref[...] + moe, g_ref[0], b_ref[0])
    x2_ref[...] = x2

    lg = jnp.dot(x2.astype(jnp.bfloat16), wr_ref[...],
                 preferred_element_type=jnp.float32) + br_ref[0]
    lane = jax.lax.broadcasted_iota(jnp.int32, lg.shape, 1)
    l8 = jnp.where(lane < N_EXPERTS, lg, _NEG)
    q0, q1, a0, a1 = _top2(l8)
    oh0 = (lane == a0).astype(jnp.float32)
    oh1 = (lane == a1).astype(jnp.float32)
    accf_ref[...] += jnp.sum(oh0 + oh1, axis=0, keepdims=True)
    accp_ref[...] += jnp.sum(q0 * oh0 + q1 * oh1, axis=0, keepdims=True)

    @pl.when(t == (S // S_TILE) - 1)
    def _():
        f = accf_ref[...] / float(S)
        p = accp_ref[...] / float(S)
        aux = jnp.float32(N_EXPERTS) * jnp.sum(f * p)
        aux_ref[...] = jnp.full((8, 128), aux, jnp.float32)


def _final(h, og, posmat, gates, ln_g, ln_b, Wr_pad, br_r):
    return pl.pallas_call(
        _final_body,
        grid=(S // S_TILE,),
        in_specs=[
            pl.BlockSpec((S_TILE, D_MODEL), lambda t: (t, 0)),
            pl.BlockSpec((MAX_ROWS, D_MODEL), lambda t: (0, 0)),
            pl.BlockSpec((S_TILE, 128), lambda t: (t, 0)),
            pl.BlockSpec((S_TILE, 128), lambda t: (t + S // S_TILE, 0)),
            pl.BlockSpec((S_TILE, 128), lambda t: (t, 0)),
            pl.BlockSpec((8, D_MODEL), lambda t: (0, 0)),
            pl.BlockSpec((8, D_MODEL), lambda t: (0, 0)),
            pl.BlockSpec((D_MODEL, 128), lambda t: (0, 0)),
            pl.BlockSpec((8, 128), lambda t: (0, 0)),
        ],
        out_specs=[
            pl.BlockSpec((S_TILE, D_MODEL), lambda t: (t, 0)),
            pl.BlockSpec((8, 128), lambda t: (0, 0)),
        ],
        out_shape=[
            jax.ShapeDtypeStruct((S, D_MODEL), jnp.float32),
            jax.ShapeDtypeStruct((8, 128), jnp.float32),
        ],
        scratch_shapes=[
            pltpu.VMEM((1, 128), jnp.float32),
            pltpu.VMEM((1, 128), jnp.float32),
        ],
    )(h, og, posmat, posmat, gates, ln_g, ln_b, Wr_pad, br_r)


def kernel(x, Wq, bq, Wk, bk, Wv, bv, Wo, bo, ln1_g, ln1_b, ln2_g, ln2_b,
           Wr, br, W1, b1, W2, b2):
    x2d = x.reshape(S, D_MODEL)

    Wr_pad = jnp.pad(Wr, ((0, 0), (0, 128 - N_EXPERTS)))
    br_pad = jnp.pad(br.reshape(1, N_EXPERTS),
                     ((0, 0), (0, 128 - N_EXPERTS)))
    br_r = jnp.broadcast_to(br_pad, (8, 128))

    def bcast8(v):
        return jnp.broadcast_to(v.reshape(1, D_MODEL), (8, D_MODEL))

    attn = _attention(x2d.astype(jnp.bfloat16), Wq, Wk, Wv, bq, bk, bv)
    h, h_bf, logits = _postattn(attn, x2d, Wo.astype(jnp.bfloat16),
                                bcast8(bo), bcast8(ln1_g), bcast8(ln1_b),
                                Wr_pad.astype(jnp.bfloat16))
    logits = logits + br_pad

    gates, posmat, temat = _route_meta(logits)
    te = temat[:MAX_TILES, 0]

    W1b = W1.astype(jnp.bfloat16)
    W2b = W2.astype(jnp.bfloat16)
    og = _moe_ffn(te, posmat, h_bf, W1b, b1.reshape(N_EXPERTS, 1, D_FF),
                  W2b, b2.reshape(N_EXPERTS, 1, D_MODEL))

    x2, auxm = _final(h, og, posmat, gates, bcast8(ln2_g), bcast8(ln2_b),
                      Wr_pad.astype(jnp.bfloat16), br_r)
    aux = auxm[0, 0]
    return (x2.reshape(1, S, D_MODEL), aux)
